# baseline (speedup 1.0000x reference)
import jax
import jax.numpy as jnp
import numpy as np
from jax.experimental import pallas as pl

BF = jnp.bfloat16
F32 = jnp.float32

G = 56
T = G * G
PAD = G + 1
K = 512
D = 64

_j = np.arange(T) % G
_MASK_L = (_j >= 1).astype(np.float32).reshape(T, 1)
_MASK_R = (_j <= G - 2).astype(np.float32).reshape(T, 1)


def _shift(hp, t):
    return hp[PAD + t: PAD + t + T, :]


def _conv3x3(h_bf, w_ref, mask_l, mask_r):
    hp = jnp.pad(h_bf, ((PAD, PAD), (0, 0)))
    masks = {-1: mask_l, 1: mask_r}
    cols = []
    for dy in (-1, 0, 1):
        for dx in (-1, 0, 1):
            c = _shift(hp, dy * G + dx)
            if dx in masks:
                c = c * masks[dx]
            cols.append(c)
    patches = jnp.concatenate(cols, axis=1)
    return jnp.dot(patches, w_ref[...], preferred_element_type=F32)


def _vq_kernel(p_ref, w1_ref, b1_ref, w2_ref, b2_ref, w3_ref, b3_ref,
               ra_ref, rba_ref, rb_ref, rbb_ref,
               sa_ref, sba_ref, sb_ref, sbb_ref,
               w4_ref, b4_ref, ct_ref, cn_ref, cb_ref,
               ml_ref, mr_ref, out_ref):
    mask_l = ml_ref[...]
    mask_r = mr_ref[...]

    h1 = jnp.dot(p_ref[0].astype(BF), w1_ref[...], preferred_element_type=F32)
    h1 = jnp.maximum(h1 + b1_ref[...], 0.0).astype(BF)

    hps = [jnp.pad(h1[:, bi * 128:bi * 128 + 64], ((PAD, PAD), (0, 0)))
           for bi in range(4)]
    masks = {-1: mask_l, 1: mask_r}
    cols = []
    for ky in range(4):
        q = ky - 1
        u = q % 2
        a = (q - u) // 2
        for kx in range(4):
            r = kx - 1
            v = r % 2
            b = (r - v) // 2
            c = _shift(hps[u * 2 + v], a * G + b)
            if b in masks:
                c = c * masks[b]
            cols.append(c)
    patches = jnp.concatenate(cols, axis=1)
    h2 = jnp.dot(patches, w2_ref[...], preferred_element_type=F32)
    h2 = jnp.maximum(h2 + b2_ref[...], 0.0)

    h3 = _conv3x3(h2.astype(BF), w3_ref, mask_l, mask_r) + b3_ref[...]

    r = jnp.maximum(h3, 0.0).astype(BF)
    r = _conv3x3(r, ra_ref, mask_l, mask_r) + rba_ref[...]
    r = jnp.maximum(r, 0.0).astype(BF)
    r = jnp.dot(r, rb_ref[...], preferred_element_type=F32) + rbb_ref[...]
    h4 = h3 + r

    r = jnp.maximum(h4, 0.0).astype(BF)
    r = _conv3x3(r, sa_ref, mask_l, mask_r) + sba_ref[...]
    r = jnp.maximum(r, 0.0).astype(BF)
    r = jnp.dot(r, sb_ref[...], preferred_element_type=F32) + sbb_ref[...]
    h5 = h4 + r

    z = jnp.dot(jnp.maximum(h5, 0.0).astype(BF), w4_ref[...],
                preferred_element_type=F32) + b4_ref[...]

    s = jnp.dot(z.astype(BF), ct_ref[...], preferred_element_type=F32)
    z2 = jnp.sum(z * z, axis=1, keepdims=True)
    d = (z2 + cn_ref[...]) - 2.0 * s
    m = jnp.min(d, axis=1, keepdims=True)
    iota = jax.lax.broadcasted_iota(jnp.int32, (T, K), 1).astype(F32)
    idx = jnp.min(jnp.where(d == m, iota, 1e9), axis=1, keepdims=True)
    onehot = (iota == idx).astype(BF)
    out_ref[0] = jnp.dot(onehot, cb_ref[...], preferred_element_type=F32)


def kernel(x, w1, b1, w2, b2, w3, b3, rw1a, rb1a, rw1b, rb1b, rw2a, rb2a,
           rw2b, rb2b, w4, b4, codebook):
    bn = x.shape[0]

    sel = np.zeros((64, 1, 6, 6), np.float32)
    for u in range(2):
        for v in range(2):
            for ky in range(4):
                for kx in range(4):
                    sel[(u * 2 + v) * 16 + ky * 4 + kx, 0,
                        2 * u + ky, 2 * v + kx] = 1.0
    p = jax.lax.conv_general_dilated(
        x.astype(BF), jnp.asarray(sel, BF), (4, 4), [(1, 1), (1, 1)],
        dimension_numbers=('NCHW', 'OIHW', 'NHWC'),
        preferred_element_type=F32)
    p = p.reshape(bn, T, 64)

    w1r = w1.reshape(64, 16).T
    w1b = jnp.concatenate(
        [jnp.pad(w1r, ((uv * 16, 48 - uv * 16), (0, 64))) for uv in range(4)],
        axis=1).astype(BF)
    b1b = jnp.concatenate(
        [jnp.pad(b1.reshape(1, 64), ((0, 0), (0, 64)))] * 4, axis=1)
    w2r = w2.transpose(2, 3, 1, 0).reshape(16 * 64, 128).astype(BF)
    w3r = w3.transpose(2, 3, 1, 0).reshape(9 * 128, 128).astype(BF)
    rar = rw1a.transpose(2, 3, 1, 0).reshape(9 * 128, 32).astype(BF)
    rbr = rw1b[:, :, 0, 0].T.astype(BF)
    sar = rw2a.transpose(2, 3, 1, 0).reshape(9 * 128, 32).astype(BF)
    sbr = rw2b[:, :, 0, 0].T.astype(BF)
    w4r = w4[:, :, 0, 0].T.astype(BF)

    ct = codebook.T.astype(BF)
    cn = jnp.sum(codebook * codebook, axis=1).reshape(1, K)
    cb = codebook.astype(BF)

    b2r, b3r, rbar, rbbr, sbar, sbbr, b4r = (
        v.reshape(1, -1) for v in (b2, b3, rb1a, rb1b, rb2a, rb2b, b4))

    mask_l = jnp.asarray(_MASK_L).astype(BF)
    mask_r = jnp.asarray(_MASK_R).astype(BF)

    args = (p, w1b, b1b, w2r, b2r, w3r, b3r, rar, rbar, rbr, rbbr,
            sar, sbar, sbr, sbbr, w4r, b4r, ct, cn, cb, mask_l, mask_r)

    def full(arr):
        return pl.BlockSpec(arr.shape, lambda i, n=arr.ndim: (0,) * n)

    in_specs = [pl.BlockSpec((1, T, 64), lambda i: (i, 0, 0))]
    in_specs += [full(a) for a in args[1:]]

    out = pl.pallas_call(
        _vq_kernel,
        grid=(bn,),
        in_specs=in_specs,
        out_specs=pl.BlockSpec((1, T, D), lambda i: (i, 0, 0)),
        out_shape=jax.ShapeDtypeStruct((bn, T, D), F32),
    )(*args)
    return out.reshape(bn, G, G, D)


# speedup vs baseline: 3.5543x; 3.5543x over previous
import jax
import jax.numpy as jnp
import numpy as np
from jax.experimental import pallas as pl

BF = jnp.bfloat16
F32 = jnp.float32

G = 56
T = G * G
PAD = G + 1
K = 512
D = 64

_j = np.arange(T) % G
_MASK_L = (_j >= 1).astype(np.float32).reshape(T, 1)
_MASK_R = (_j <= G - 2).astype(np.float32).reshape(T, 1)


def _shift(hp, t):
    return hp[PAD + t: PAD + t + T, :]


def _conv3x3(h_bf, w_ref, mask_l, mask_r):
    hp = jnp.pad(h_bf, ((PAD, PAD), (0, 0)))
    masks = {-1: mask_l, 1: mask_r}
    cols = []
    for dy in (-1, 0, 1):
        for dx in (-1, 0, 1):
            c = _shift(hp, dy * G + dx)
            if dx in masks:
                c = c * masks[dx]
            cols.append(c)
    patches = jnp.concatenate(cols, axis=1)
    return jnp.dot(patches, w_ref[...], preferred_element_type=F32)


def _vq_kernel(p_ref, w1_ref, b1_ref, w2_ref, b2_ref, w3_ref, b3_ref,
               ra_ref, rba_ref, rb_ref, rbb_ref,
               sa_ref, sba_ref, sb_ref, sbb_ref,
               w4_ref, b4_ref, ct_ref, cn_ref, cb_ref,
               ml_ref, mr_ref, out_ref):
    mask_l = ml_ref[...]
    mask_r = mr_ref[...]

    h1 = jnp.dot(p_ref[0].astype(BF), w1_ref[...], preferred_element_type=F32)
    h1 = jnp.maximum(h1 + b1_ref[...], 0.0).astype(BF)

    hps = [jnp.pad(h1[:, bi * 128:bi * 128 + 64], ((PAD, PAD), (0, 0)))
           for bi in range(4)]
    masks = {-1: mask_l, 1: mask_r}
    cols = []
    for ky in range(4):
        q = ky - 1
        u = q % 2
        a = (q - u) // 2
        for kx in range(4):
            r = kx - 1
            v = r % 2
            b = (r - v) // 2
            c = _shift(hps[u * 2 + v], a * G + b)
            if b in masks:
                c = c * masks[b]
            cols.append(c)
    patches = jnp.concatenate(cols, axis=1)
    h2 = jnp.dot(patches, w2_ref[...], preferred_element_type=F32)
    h2 = jnp.maximum(h2 + b2_ref[...], 0.0)

    h3 = _conv3x3(h2.astype(BF), w3_ref, mask_l, mask_r) + b3_ref[...]

    r = jnp.maximum(h3, 0.0).astype(BF)
    r = _conv3x3(r, ra_ref, mask_l, mask_r) + rba_ref[...]
    r = jnp.maximum(r, 0.0).astype(BF)
    r = jnp.dot(r, rb_ref[...], preferred_element_type=F32) + rbb_ref[...]
    h4 = h3 + r

    r = jnp.maximum(h4, 0.0).astype(BF)
    r = _conv3x3(r, sa_ref, mask_l, mask_r) + sba_ref[...]
    r = jnp.maximum(r, 0.0).astype(BF)
    r = jnp.dot(r, sb_ref[...], preferred_element_type=F32) + sbb_ref[...]
    h5 = h4 + r

    z = jnp.dot(jnp.maximum(h5, 0.0).astype(BF), w4_ref[...],
                preferred_element_type=F32) + b4_ref[...]

    s = jnp.dot(z.astype(BF), ct_ref[...], preferred_element_type=F32)
    z2 = jnp.sum(z * z, axis=1, keepdims=True)
    d = (z2 + cn_ref[...]) - 2.0 * s
    m = jnp.min(d, axis=1, keepdims=True)
    iota = jax.lax.broadcasted_iota(jnp.int32, (T, K), 1).astype(F32)
    idx = jnp.min(jnp.where(d == m, iota, 1e9), axis=1, keepdims=True)
    onehot = (iota == idx).astype(BF)
    out_ref[0] = jnp.dot(onehot, cb_ref[...], preferred_element_type=F32)


def kernel(x, w1, b1, w2, b2, w3, b3, rw1a, rb1a, rw1b, rb1b, rw2a, rb2a,
           rw2b, rb2b, w4, b4, codebook):
    bn = x.shape[0]

    sel = np.zeros((6, 3, 4, 64), np.float32)
    for u in range(2):
        for v in range(2):
            for ky in range(4):
                for kx in range(4):
                    q = (u * 2 + v) * 16 + ky * 4 + kx
                    c = 2 * v + kx - 1
                    sel[2 * u + ky, c // 4 + 1, c % 4, q] = 1.0
    p = jax.lax.conv_general_dilated(
        x.reshape(bn, 224, G, 4).astype(BF), jnp.asarray(sel, BF),
        (4, 1), [(1, 1), (1, 1)],
        dimension_numbers=('NHWC', 'HWIO', 'NHWC'),
        preferred_element_type=F32)
    p = p.reshape(bn, T, 64)

    w1r = w1.reshape(64, 16).T
    w1b = jnp.concatenate(
        [jnp.pad(w1r, ((uv * 16, 48 - uv * 16), (0, 64))) for uv in range(4)],
        axis=1).astype(BF)
    b1b = jnp.concatenate(
        [jnp.pad(b1.reshape(1, 64), ((0, 0), (0, 64)))] * 4, axis=1)
    w2r = w2.transpose(2, 3, 1, 0).reshape(16 * 64, 128).astype(BF)
    w3r = w3.transpose(2, 3, 1, 0).reshape(9 * 128, 128).astype(BF)
    rar = rw1a.transpose(2, 3, 1, 0).reshape(9 * 128, 32).astype(BF)
    rbr = rw1b[:, :, 0, 0].T.astype(BF)
    sar = rw2a.transpose(2, 3, 1, 0).reshape(9 * 128, 32).astype(BF)
    sbr = rw2b[:, :, 0, 0].T.astype(BF)
    w4r = w4[:, :, 0, 0].T.astype(BF)

    ct = codebook.T.astype(BF)
    cn = jnp.sum(codebook * codebook, axis=1).reshape(1, K)
    cb = codebook.astype(BF)

    b2r, b3r, rbar, rbbr, sbar, sbbr, b4r = (
        v.reshape(1, -1) for v in (b2, b3, rb1a, rb1b, rb2a, rb2b, b4))

    mask_l = jnp.asarray(_MASK_L).astype(BF)
    mask_r = jnp.asarray(_MASK_R).astype(BF)

    args = (p, w1b, b1b, w2r, b2r, w3r, b3r, rar, rbar, rbr, rbbr,
            sar, sbar, sbr, sbbr, w4r, b4r, ct, cn, cb, mask_l, mask_r)

    def full(arr):
        return pl.BlockSpec(arr.shape, lambda i, n=arr.ndim: (0,) * n)

    in_specs = [pl.BlockSpec((1, T, 64), lambda i: (i, 0, 0))]
    in_specs += [full(a) for a in args[1:]]

    out = pl.pallas_call(
        _vq_kernel,
        grid=(bn,),
        in_specs=in_specs,
        out_specs=pl.BlockSpec((1, T, D), lambda i: (i, 0, 0)),
        out_shape=jax.ShapeDtypeStruct((bn, T, D), F32),
    )(*args)
    return out.reshape(bn, G, G, D)
